# baseline (speedup 1.0000x reference)
import functools

import jax
import jax.numpy as jnp
from jax import lax
from jax.experimental import pallas as pl
from jax.experimental.pallas import tpu as pltpu
from jax.experimental.pallas import tpu_sc as plsc

N = 10000
E = 320000
H = 128
EPS = 1e-5

NC = 2
NS = 16
NW = NC * NS
CH = 128
NCHUNK = E // CH
TRIPS = -(-NCHUNK // NW)
NPAD = 10240
DEG_SLAB = NPAD // NS
ROW_SLAB = NPAD // NS

def _sc_mesh():
    return plsc.VectorSubcoreMesh(
        core_axis_name="c", subcore_axis_name="s",
        num_cores=NC, num_subcores=NS)


def _deg_body(ei_hbm, zeros_hbm, degp_hbm, didx, didx1, ones, acc,
              semd0, semd1):
    c = lax.axis_index("c")
    s = lax.axis_index("s")
    wid = s * NC + c
    for j in range(CH // 16):
        ones[pl.ds(j * 16, 16)] = jnp.full((16,), 1.0, jnp.float32)
    pltpu.sync_copy(zeros_hbm.at[pl.ds(s * DEG_SLAB, DEG_SLAB)],
                    acc.at[pl.ds(s * DEG_SLAB, DEG_SLAB)])
    plsc.subcore_barrier()

    base = wid * WPW * CH
    pltpu.async_copy(ei_hbm.at[:, pl.ds(base, CH)], didx, semd0)
    pltpu.async_copy(ei_hbm.at[:, pl.ds(base + CH, CH)], didx1, semd1)

    def body(k, carry):
        pltpu.make_async_copy(ei_hbm.at[:, pl.ds(0, CH)], didx, semd0).wait()
        pltpu.sync_copy(ones, acc.at[didx.at[1]], add=True)

        @pl.when(2 * k + 2 < WPW)
        def _():
            pltpu.async_copy(ei_hbm.at[:, pl.ds(base + (2 * k + 2) * CH, CH)],
                             didx, semd0)
        pltpu.make_async_copy(ei_hbm.at[:, pl.ds(0, CH)], didx1, semd1).wait()
        pltpu.sync_copy(ones, acc.at[didx1.at[1]], add=True)

        @pl.when(2 * k + 3 < WPW)
        def _():
            pltpu.async_copy(ei_hbm.at[:, pl.ds(base + (2 * k + 3) * CH, CH)],
                             didx1, semd1)
        return carry

    lax.fori_loop(0, PAIRS, body, 0)

    @pl.when(wid < NCHUNK - WPW * NW)
    def _():
        ex = WPW * NW + wid
        pltpu.sync_copy(ei_hbm.at[:, pl.ds(ex * CH, CH)], didx)
        pltpu.sync_copy(ones, acc.at[didx.at[1]], add=True)

    plsc.subcore_barrier()
    pltpu.sync_copy(acc.at[pl.ds(s * DEG_SLAB, DEG_SLAB)],
                    degp_hbm.at[pl.ds(c * NPAD + s * DEG_SLAB, DEG_SLAB)])


WPW = NCHUNK // NW
PAIRS = WPW // 2
N_TAIL = N - (NS - 1) * ROW_SLAB


def _scatter_body(g_hbm, ei_hbm, zeros_hbm, accp_hbm,
                  ebuf0, ebuf1, rows0, rows1,
                  acc, sem0, sem1, semd0, semd1):
    c = lax.axis_index("c")
    s = lax.axis_index("s")
    wid = s * NC + c
    base = wid * WPW * CH

    @pl.when(jnp.logical_and(c == 0, s < NS - 1))
    def _():
        pltpu.sync_copy(g_hbm.at[pl.ds(s * ROW_SLAB, ROW_SLAB)],
                        acc.at[pl.ds(s * ROW_SLAB, ROW_SLAB)])

    @pl.when(jnp.logical_and(c == 0, s == NS - 1))
    def _():
        pltpu.sync_copy(g_hbm.at[pl.ds((NS - 1) * ROW_SLAB, N_TAIL)],
                        acc.at[pl.ds((NS - 1) * ROW_SLAB, N_TAIL)])

    @pl.when(c == 1)
    def _():
        pltpu.sync_copy(zeros_hbm.at[pl.ds(s * ROW_SLAB, ROW_SLAB)],
                        acc.at[pl.ds(s * ROW_SLAB, ROW_SLAB)])

    plsc.subcore_barrier()

    pltpu.async_copy(ei_hbm.at[:, pl.ds(base, CH)], ebuf0, semd0)
    pltpu.async_copy(ei_hbm.at[:, pl.ds(base + CH, CH)], ebuf1, semd1)
    pltpu.make_async_copy(ei_hbm.at[:, pl.ds(0, CH)], ebuf0, semd0).wait()
    pltpu.async_copy(g_hbm.at[ebuf0.at[0]], rows0, sem0)

    def body(k, carry):
        pltpu.make_async_copy(ei_hbm.at[:, pl.ds(0, CH)], ebuf1, semd1).wait()
        pltpu.async_copy(g_hbm.at[ebuf1.at[0]], rows1, sem1)
        pltpu.make_async_copy(g_hbm.at[pl.ds(0, CH)], rows0, sem0).wait()
        pltpu.sync_copy(rows0, acc.at[ebuf0.at[1]], add=True)

        @pl.when(2 * k + 2 < WPW)
        def _():
            pltpu.async_copy(ei_hbm.at[:, pl.ds(base + (2 * k + 2) * CH, CH)],
                             ebuf0, semd0)
        pltpu.make_async_copy(g_hbm.at[pl.ds(0, CH)], rows1, sem1).wait()
        pltpu.sync_copy(rows1, acc.at[ebuf1.at[1]], add=True)

        @pl.when(2 * k + 3 < WPW)
        def _():
            pltpu.async_copy(ei_hbm.at[:, pl.ds(base + (2 * k + 3) * CH, CH)],
                             ebuf1, semd1)

        @pl.when(2 * k + 2 < WPW)
        def _():
            pltpu.make_async_copy(ei_hbm.at[:, pl.ds(0, CH)], ebuf0,
                                  semd0).wait()
            pltpu.async_copy(g_hbm.at[ebuf0.at[0]], rows0, sem0)
        return carry

    lax.fori_loop(0, PAIRS, body, 0)

    @pl.when(wid < NCHUNK - WPW * NW)
    def _():
        ex = WPW * NW + wid
        pltpu.sync_copy(ei_hbm.at[:, pl.ds(ex * CH, CH)], ebuf0)
        pltpu.async_copy(g_hbm.at[ebuf0.at[0]], rows0, sem0).wait()
        pltpu.sync_copy(rows0, acc.at[ebuf0.at[1]], add=True)

    plsc.subcore_barrier()
    pltpu.sync_copy(acc.at[pl.ds(s * ROW_SLAB, ROW_SLAB)],
                    accp_hbm.at[c, pl.ds(s * ROW_SLAB, ROW_SLAB)])


@functools.cache
def _sc_kernels():
    deg_kernel = pl.kernel(
        _deg_body,
        out_type=jax.ShapeDtypeStruct((NC * NPAD,), jnp.float32),
        mesh=_sc_mesh(),
        scratch_types=[
            pltpu.VMEM((2, CH), jnp.int32),
            pltpu.VMEM((2, CH), jnp.int32),
            pltpu.VMEM((CH,), jnp.float32),
            pltpu.VMEM_SHARED((NPAD,), jnp.float32),
            pltpu.SemaphoreType.DMA,
            pltpu.SemaphoreType.DMA,
        ],
    )
    scatter_kernel = pl.kernel(
        _scatter_body,
        out_type=jax.ShapeDtypeStruct((NC, NPAD, H), jnp.float32),
        mesh=_sc_mesh(),
        scratch_types=[
            pltpu.VMEM((2, CH), jnp.int32),
            pltpu.VMEM((2, CH), jnp.int32),
            pltpu.VMEM((CH, H), jnp.float32),
            pltpu.VMEM((CH, H), jnp.float32),
            pltpu.VMEM_SHARED((NPAD, H), jnp.float32),
            pltpu.SemaphoreType.DMA,
            pltpu.SemaphoreType.DMA,
            pltpu.SemaphoreType.DMA,
            pltpu.SemaphoreType.DMA,
        ],
    )
    return deg_kernel, scatter_kernel


R = 2000
GRID = N // R


def _dinv_rows(deg_ref):
    deg = deg_ref[:, 0] + deg_ref[:, 1] + 1.0
    return lax.rsqrt(deg)[:, None]


def _tc_in_body(x_ref, w0_ref, b0_ref, c0_ref, deg_ref, g_ref):
    h = jnp.dot(x_ref[...], w0_ref[...], preferred_element_type=jnp.float32)
    h = jnp.maximum(h + b0_ref[...], 0.0)
    g_ref[...] = jnp.dot(_dinv_rows(deg_ref) * h, c0_ref[...],
                         preferred_element_type=jnp.float32)


def _tc_post_body(acca_ref, accb_ref, deg_ref, b_ref,
                  z_ref, s1_ref, s2_ref):
    i = pl.program_id(0)
    z = _dinv_rows(deg_ref) * (acca_ref[0] + accb_ref[0])
    z = z + b_ref[...]
    z_ref[...] = z

    @pl.when(i == 0)
    def _():
        s1_ref[...] = jnp.zeros_like(s1_ref)
        s2_ref[...] = jnp.zeros_like(s2_ref)
    s1_ref[...] += jnp.sum(z, axis=0, keepdims=True)
    s2_ref[...] += jnp.sum(z * z, axis=0, keepdims=True)


def _bn_relu(z_ref, s1_ref, s2_ref, bg_ref, bb_ref):
    mean = s1_ref[...] * (1.0 / N)
    var = s2_ref[...] * (1.0 / N) - mean * mean
    zn = (z_ref[...] - mean) * lax.rsqrt(var + EPS) * bg_ref[...] + bb_ref[...]
    return jnp.maximum(zn, 0.0)


def _tc_mid_body(z_ref, s1_ref, s2_ref, bg_ref, bb_ref, deg_ref, c1_ref,
                 g_ref):
    h = _bn_relu(z_ref, s1_ref, s2_ref, bg_ref, bb_ref)
    g_ref[...] = jnp.dot(_dinv_rows(deg_ref) * h, c1_ref[...],
                         preferred_element_type=jnp.float32)


def _tc_out_body(z_ref, s1_ref, s2_ref, bg_ref, bb_ref, w_ref, b_ref, o_ref):
    h = _bn_relu(z_ref, s1_ref, s2_ref, bg_ref, bb_ref)
    o_ref[...] = jnp.dot(h, w_ref[...],
                         preferred_element_type=jnp.float32) + b_ref[...]


_row_spec = pl.BlockSpec((R, H), lambda i: (i, 0))
_w_spec = pl.BlockSpec((H, H), lambda i: (0, 0))
_v_spec = pl.BlockSpec((1, H), lambda i: (0, 0))
_deg_spec = pl.BlockSpec((R, 2), lambda i: (i, 0))

_f32 = jnp.float32

_tc_in = pl.pallas_call(
    _tc_in_body,
    grid=(GRID,),
    in_specs=[_row_spec, _w_spec, _v_spec, _w_spec, _deg_spec],
    out_specs=_row_spec,
    out_shape=jax.ShapeDtypeStruct((N, H), _f32),
)

_acc0_spec = pl.BlockSpec((1, R, H), lambda i: (0, i, 0))
_acc1_spec = pl.BlockSpec((1, R, H), lambda i: (1, i, 0))

_tc_post = pl.pallas_call(
    _tc_post_body,
    grid=(GRID,),
    in_specs=[_acc0_spec, _acc1_spec, _deg_spec, _v_spec],
    out_specs=[_row_spec, _v_spec, _v_spec],
    out_shape=[jax.ShapeDtypeStruct((N, H), _f32),
               jax.ShapeDtypeStruct((1, H), _f32),
               jax.ShapeDtypeStruct((1, H), _f32)],
)

_tc_mid = pl.pallas_call(
    _tc_mid_body,
    grid=(GRID,),
    in_specs=[_row_spec, _v_spec, _v_spec, _v_spec, _v_spec, _deg_spec,
              _w_spec],
    out_specs=_row_spec,
    out_shape=jax.ShapeDtypeStruct((N, H), _f32),
)

_tc_out = pl.pallas_call(
    _tc_out_body,
    grid=(GRID,),
    in_specs=[_row_spec, _v_spec, _v_spec, _v_spec, _v_spec, _w_spec,
              _v_spec],
    out_specs=_row_spec,
    out_shape=jax.ShapeDtypeStruct((N, H), _f32),
)


def kernel(x, edge_index, edge_attr, lin0_w, lin0_b, conv0_w, conv0_b,
           bn0_g, bn0_b, conv1_w, conv1_b, bn1_g, bn1_b, lin1_w, lin1_b):
    ei = edge_index.astype(jnp.int32)
    zeros_deg = jnp.zeros((NPAD,), _f32)
    zeros_acc = jnp.zeros((NPAD, H), _f32)

    _deg_kernel, _scatter_kernel = _sc_kernels()
    degp = _deg_kernel(ei, zeros_deg).reshape(NC, NPAD)
    deg2 = degp[:, :N].T

    b0 = lin0_b.reshape(1, H)
    cb0 = conv0_b.reshape(1, H)
    cb1 = conv1_b.reshape(1, H)
    g0 = _tc_in(x, lin0_w, b0, conv0_w, deg2)

    accp0 = _scatter_kernel(g0, ei, zeros_acc)
    z0, s01, s02 = _tc_post(accp0, accp0, deg2, cb0)

    g1 = _tc_mid(z0, s01, s02, bn0_g.reshape(1, H), bn0_b.reshape(1, H),
                 deg2, conv1_w)

    accp1 = _scatter_kernel(g1, ei, zeros_acc)
    z1, s11, s12 = _tc_post(accp1, accp1, deg2, cb1)

    out = _tc_out(z1, s11, s12, bn1_g.reshape(1, H), bn1_b.reshape(1, H),
                  lin1_w, lin1_b.reshape(1, H))
    return out


# speedup vs baseline: 1.1795x; 1.1795x over previous
import functools

import jax
import jax.numpy as jnp
from jax import lax
from jax.experimental import pallas as pl
from jax.experimental.pallas import tpu as pltpu
from jax.experimental.pallas import tpu_sc as plsc

N = 10000
E = 320000
H = 128
EPS = 1e-5

NC = 2
NS = 16
NW = NC * NS
CH = 128
NCHUNK = E // CH
TRIPS = -(-NCHUNK // NW)
NPAD = 10240
DEG_SLAB = NPAD // NS
ROW_SLAB = NPAD // NS

def _sc_mesh():
    return plsc.VectorSubcoreMesh(
        core_axis_name="c", subcore_axis_name="s",
        num_cores=NC, num_subcores=NS)


def _deg_body(ei_hbm, zeros_hbm, degp_hbm, didx, didx1, ones, acc,
              semd0, semd1):
    c = lax.axis_index("c")
    s = lax.axis_index("s")
    wid = s * NC + c
    for j in range(CH // 16):
        ones[pl.ds(j * 16, 16)] = jnp.full((16,), 1.0, jnp.float32)
    pltpu.sync_copy(zeros_hbm.at[pl.ds(s * DEG_SLAB, DEG_SLAB)],
                    acc.at[pl.ds(s * DEG_SLAB, DEG_SLAB)])
    plsc.subcore_barrier()

    base = wid * WPW * CH
    pltpu.async_copy(ei_hbm.at[:, pl.ds(base, CH)], didx, semd0)
    pltpu.async_copy(ei_hbm.at[:, pl.ds(base + CH, CH)], didx1, semd1)

    def body(k, carry):
        pltpu.make_async_copy(ei_hbm.at[:, pl.ds(0, CH)], didx, semd0).wait()
        pltpu.sync_copy(ones, acc.at[didx.at[1]], add=True)

        @pl.when(2 * k + 2 < WPW)
        def _():
            pltpu.async_copy(ei_hbm.at[:, pl.ds(base + (2 * k + 2) * CH, CH)],
                             didx, semd0)
        pltpu.make_async_copy(ei_hbm.at[:, pl.ds(0, CH)], didx1, semd1).wait()
        pltpu.sync_copy(ones, acc.at[didx1.at[1]], add=True)

        @pl.when(2 * k + 3 < WPW)
        def _():
            pltpu.async_copy(ei_hbm.at[:, pl.ds(base + (2 * k + 3) * CH, CH)],
                             didx1, semd1)
        return carry

    lax.fori_loop(0, PAIRS, body, 0)

    @pl.when(wid < NCHUNK - WPW * NW)
    def _():
        ex = WPW * NW + wid
        pltpu.sync_copy(ei_hbm.at[:, pl.ds(ex * CH, CH)], didx)
        pltpu.sync_copy(ones, acc.at[didx.at[1]], add=True)

    plsc.subcore_barrier()
    pltpu.sync_copy(acc.at[pl.ds(s * DEG_SLAB, DEG_SLAB)],
                    degp_hbm.at[pl.ds(c * NPAD + s * DEG_SLAB, DEG_SLAB)])


WPW = NCHUNK // NW
PAIRS = WPW // 2
N_TAIL = N - (NS - 1) * ROW_SLAB


def _scatter_body(g_hbm, ei_hbm, zeros_hbm, accp_hbm,
                  sidx_flat, ebuf0, ebuf1, rows0, rows1,
                  acc, sem0, sem1, semd0, semd1):
    c = lax.axis_index("c")
    s = lax.axis_index("s")
    wid = s * NC + c
    base = wid * WPW * CH

    @pl.when(jnp.logical_and(c == 0, s < NS - 1))
    def _():
        pltpu.sync_copy(g_hbm.at[pl.ds(s * ROW_SLAB, ROW_SLAB)],
                        acc.at[pl.ds(s * ROW_SLAB, ROW_SLAB)])

    @pl.when(jnp.logical_and(c == 0, s == NS - 1))
    def _():
        pltpu.sync_copy(g_hbm.at[pl.ds((NS - 1) * ROW_SLAB, N_TAIL)],
                        acc.at[pl.ds((NS - 1) * ROW_SLAB, N_TAIL)])

    @pl.when(c == 1)
    def _():
        pltpu.sync_copy(zeros_hbm.at[pl.ds(s * ROW_SLAB, ROW_SLAB)],
                        acc.at[pl.ds(s * ROW_SLAB, ROW_SLAB)])

    pltpu.sync_copy(ei_hbm.at[0, pl.ds(base, WPW * CH)], sidx_flat)
    plsc.subcore_barrier()

    pltpu.async_copy(ei_hbm.at[:, pl.ds(base, CH)], ebuf0, semd0)
    pltpu.async_copy(g_hbm.at[sidx_flat.at[pl.ds(0, CH)]], rows0, sem0)

    def body(k, carry):
        t1 = 2 * k + 1
        pltpu.async_copy(ei_hbm.at[:, pl.ds(base + t1 * CH, CH)],
                         ebuf1, semd1)
        pltpu.async_copy(g_hbm.at[sidx_flat.at[pl.ds(t1 * CH, CH)]],
                         rows1, sem1)
        pltpu.make_async_copy(g_hbm.at[pl.ds(0, CH)], rows0, sem0).wait()
        pltpu.make_async_copy(ei_hbm.at[:, pl.ds(0, CH)], ebuf0,
                              semd0).wait()
        pltpu.sync_copy(rows0, acc.at[ebuf0.at[1]], add=True)

        @pl.when(2 * k + 2 < WPW)
        def _():
            pltpu.async_copy(ei_hbm.at[:, pl.ds(base + (2 * k + 2) * CH, CH)],
                             ebuf0, semd0)
            pltpu.async_copy(
                g_hbm.at[sidx_flat.at[pl.ds((2 * k + 2) * CH, CH)]],
                rows0, sem0)
        pltpu.make_async_copy(g_hbm.at[pl.ds(0, CH)], rows1, sem1).wait()
        pltpu.make_async_copy(ei_hbm.at[:, pl.ds(0, CH)], ebuf1,
                              semd1).wait()
        pltpu.sync_copy(rows1, acc.at[ebuf1.at[1]], add=True)
        return carry

    lax.fori_loop(0, PAIRS, body, 0)

    @pl.when(wid < NCHUNK - WPW * NW)
    def _():
        ex = WPW * NW + wid
        pltpu.sync_copy(ei_hbm.at[:, pl.ds(ex * CH, CH)], ebuf0)
        pltpu.sync_copy(ei_hbm.at[0, pl.ds(ex * CH, CH)],
                        sidx_flat.at[pl.ds(0, CH)])
        pltpu.async_copy(g_hbm.at[sidx_flat.at[pl.ds(0, CH)]],
                         rows0, sem0).wait()
        pltpu.sync_copy(rows0, acc.at[ebuf0.at[1]], add=True)

    plsc.subcore_barrier()
    pltpu.sync_copy(acc.at[pl.ds(s * ROW_SLAB, ROW_SLAB)],
                    accp_hbm.at[c, pl.ds(s * ROW_SLAB, ROW_SLAB)])


@functools.cache
def _sc_kernels():
    deg_kernel = pl.kernel(
        _deg_body,
        out_type=jax.ShapeDtypeStruct((NC * NPAD,), jnp.float32),
        mesh=_sc_mesh(),
        scratch_types=[
            pltpu.VMEM((2, CH), jnp.int32),
            pltpu.VMEM((2, CH), jnp.int32),
            pltpu.VMEM((CH,), jnp.float32),
            pltpu.VMEM_SHARED((NPAD,), jnp.float32),
            pltpu.SemaphoreType.DMA,
            pltpu.SemaphoreType.DMA,
        ],
    )
    scatter_kernel = pl.kernel(
        _scatter_body,
        out_type=jax.ShapeDtypeStruct((NC, NPAD, H), jnp.float32),
        mesh=_sc_mesh(),
        scratch_types=[
            pltpu.VMEM((WPW * CH,), jnp.int32),
            pltpu.VMEM((2, CH), jnp.int32),
            pltpu.VMEM((2, CH), jnp.int32),
            pltpu.VMEM((CH, H), jnp.float32),
            pltpu.VMEM((CH, H), jnp.float32),
            pltpu.VMEM_SHARED((NPAD, H), jnp.float32),
            pltpu.SemaphoreType.DMA,
            pltpu.SemaphoreType.DMA,
            pltpu.SemaphoreType.DMA,
            pltpu.SemaphoreType.DMA,
        ],
    )
    return deg_kernel, scatter_kernel


R = 2000
GRID = N // R


def _dinv_rows(deg_ref):
    deg = deg_ref[:, 0] + deg_ref[:, 1] + 1.0
    return lax.rsqrt(deg)[:, None]


def _tc_in_body(x_ref, w0_ref, b0_ref, c0_ref, deg_ref, g_ref):
    h = jnp.dot(x_ref[...], w0_ref[...], preferred_element_type=jnp.float32)
    h = jnp.maximum(h + b0_ref[...], 0.0)
    g_ref[...] = jnp.dot(_dinv_rows(deg_ref) * h, c0_ref[...],
                         preferred_element_type=jnp.float32)


def _tc_post_body(acca_ref, accb_ref, deg_ref, b_ref,
                  z_ref, s1_ref, s2_ref):
    i = pl.program_id(0)
    z = _dinv_rows(deg_ref) * (acca_ref[0] + accb_ref[0])
    z = z + b_ref[...]
    z_ref[...] = z

    @pl.when(i == 0)
    def _():
        s1_ref[...] = jnp.zeros_like(s1_ref)
        s2_ref[...] = jnp.zeros_like(s2_ref)
    s1_ref[...] += jnp.sum(z, axis=0, keepdims=True)
    s2_ref[...] += jnp.sum(z * z, axis=0, keepdims=True)


def _bn_relu(z_ref, s1_ref, s2_ref, bg_ref, bb_ref):
    mean = s1_ref[...] * (1.0 / N)
    var = s2_ref[...] * (1.0 / N) - mean * mean
    zn = (z_ref[...] - mean) * lax.rsqrt(var + EPS) * bg_ref[...] + bb_ref[...]
    return jnp.maximum(zn, 0.0)


def _tc_mid_body(z_ref, s1_ref, s2_ref, bg_ref, bb_ref, deg_ref, c1_ref,
                 g_ref):
    h = _bn_relu(z_ref, s1_ref, s2_ref, bg_ref, bb_ref)
    g_ref[...] = jnp.dot(_dinv_rows(deg_ref) * h, c1_ref[...],
                         preferred_element_type=jnp.float32)


def _tc_out_body(z_ref, s1_ref, s2_ref, bg_ref, bb_ref, w_ref, b_ref, o_ref):
    h = _bn_relu(z_ref, s1_ref, s2_ref, bg_ref, bb_ref)
    o_ref[...] = jnp.dot(h, w_ref[...],
                         preferred_element_type=jnp.float32) + b_ref[...]


_row_spec = pl.BlockSpec((R, H), lambda i: (i, 0))
_w_spec = pl.BlockSpec((H, H), lambda i: (0, 0))
_v_spec = pl.BlockSpec((1, H), lambda i: (0, 0))
_deg_spec = pl.BlockSpec((R, 2), lambda i: (i, 0))

_f32 = jnp.float32

_tc_in = pl.pallas_call(
    _tc_in_body,
    grid=(GRID,),
    in_specs=[_row_spec, _w_spec, _v_spec, _w_spec, _deg_spec],
    out_specs=_row_spec,
    out_shape=jax.ShapeDtypeStruct((N, H), _f32),
)

_acc0_spec = pl.BlockSpec((1, R, H), lambda i: (0, i, 0))
_acc1_spec = pl.BlockSpec((1, R, H), lambda i: (1, i, 0))

_tc_post = pl.pallas_call(
    _tc_post_body,
    grid=(GRID,),
    in_specs=[_acc0_spec, _acc1_spec, _deg_spec, _v_spec],
    out_specs=[_row_spec, _v_spec, _v_spec],
    out_shape=[jax.ShapeDtypeStruct((N, H), _f32),
               jax.ShapeDtypeStruct((1, H), _f32),
               jax.ShapeDtypeStruct((1, H), _f32)],
)

_tc_mid = pl.pallas_call(
    _tc_mid_body,
    grid=(GRID,),
    in_specs=[_row_spec, _v_spec, _v_spec, _v_spec, _v_spec, _deg_spec,
              _w_spec],
    out_specs=_row_spec,
    out_shape=jax.ShapeDtypeStruct((N, H), _f32),
)

_tc_out = pl.pallas_call(
    _tc_out_body,
    grid=(GRID,),
    in_specs=[_row_spec, _v_spec, _v_spec, _v_spec, _v_spec, _w_spec,
              _v_spec],
    out_specs=_row_spec,
    out_shape=jax.ShapeDtypeStruct((N, H), _f32),
)


def kernel(x, edge_index, edge_attr, lin0_w, lin0_b, conv0_w, conv0_b,
           bn0_g, bn0_b, conv1_w, conv1_b, bn1_g, bn1_b, lin1_w, lin1_b):
    ei = edge_index.astype(jnp.int32)
    zeros_deg = jnp.zeros((NPAD,), _f32)
    zeros_acc = jnp.zeros((NPAD, H), _f32)

    _deg_kernel, _scatter_kernel = _sc_kernels()
    degp = _deg_kernel(ei, zeros_deg).reshape(NC, NPAD)
    deg2 = degp[:, :N].T

    b0 = lin0_b.reshape(1, H)
    cb0 = conv0_b.reshape(1, H)
    cb1 = conv1_b.reshape(1, H)
    g0 = _tc_in(x, lin0_w, b0, conv0_w, deg2)

    accp0 = _scatter_kernel(g0, ei, zeros_acc)
    z0, s01, s02 = _tc_post(accp0, accp0, deg2, cb0)

    g1 = _tc_mid(z0, s01, s02, bn0_g.reshape(1, H), bn0_b.reshape(1, H),
                 deg2, conv1_w)

    accp1 = _scatter_kernel(g1, ei, zeros_acc)
    z1, s11, s12 = _tc_post(accp1, accp1, deg2, cb1)

    out = _tc_out(z1, s11, s12, bn1_g.reshape(1, H), bn1_b.reshape(1, H),
                  lin1_w, lin1_b.reshape(1, H))
    return out


# speedup vs baseline: 1.2200x; 1.0343x over previous
import functools

import jax
import jax.numpy as jnp
from jax import lax
from jax.experimental import pallas as pl
from jax.experimental.pallas import tpu as pltpu
from jax.experimental.pallas import tpu_sc as plsc

N = 10000
E = 320000
H = 128
EPS = 1e-5

NC = 2
NS = 16
NW = NC * NS
CH = 128
NCHUNK = E // CH
TRIPS = -(-NCHUNK // NW)
NPAD = 10240
DEG_SLAB = NPAD // NS
ROW_SLAB = NPAD // NS

def _sc_mesh():
    return plsc.VectorSubcoreMesh(
        core_axis_name="c", subcore_axis_name="s",
        num_cores=NC, num_subcores=NS)


def _deg_body(ei_hbm, zeros_hbm, degp_hbm, didx0, didx1, didx2, didx3,
              ones, acc, semd0, semd1, semd2, semd3):
    c = lax.axis_index("c")
    s = lax.axis_index("s")
    wid = s * NC + c
    for j in range(CH // 16):
        ones[pl.ds(j * 16, 16)] = jnp.full((16,), 1.0, jnp.float32)
    pltpu.sync_copy(zeros_hbm, acc.at[pl.ds(s * DEG_SLAB, DEG_SLAB)])
    plsc.subcore_barrier()

    base = wid * WPW * CH
    pltpu.async_copy(ei_hbm.at[:, pl.ds(base, CH)], didx0, semd0)
    pltpu.async_copy(ei_hbm.at[:, pl.ds(base + CH, CH)], didx1, semd1)
    pltpu.async_copy(ei_hbm.at[:, pl.ds(base + 2 * CH, CH)], didx2, semd2)
    pltpu.async_copy(ei_hbm.at[:, pl.ds(base + 3 * CH, CH)], didx3, semd3)

    def body(k, carry):
        t = 4 * k
        for j, (buf, sem) in enumerate([(didx0, semd0), (didx1, semd1),
                                        (didx2, semd2), (didx3, semd3)]):
            pltpu.make_async_copy(ei_hbm.at[:, pl.ds(0, CH)], buf, sem).wait()
            pltpu.sync_copy(ones, acc.at[buf.at[1]], add=True)

            @pl.when(t + j + 4 < WPW)
            def _():
                pltpu.async_copy(
                    ei_hbm.at[:, pl.ds(base + (t + j + 4) * CH, CH)],
                    buf, sem)
        return carry

    lax.fori_loop(0, WPW // 4, body, 0)

    for j, (buf, sem) in enumerate([(didx0, semd0), (didx1, semd1)]):
        pltpu.make_async_copy(ei_hbm.at[:, pl.ds(0, CH)], buf, sem).wait()
        pltpu.sync_copy(ones, acc.at[buf.at[1]], add=True)

    @pl.when(wid < NCHUNK - WPW * NW)
    def _():
        ex = WPW * NW + wid
        pltpu.sync_copy(ei_hbm.at[:, pl.ds(ex * CH, CH)], didx0)
        pltpu.sync_copy(ones, acc.at[didx0.at[1]], add=True)

    plsc.subcore_barrier()
    pltpu.sync_copy(acc.at[pl.ds(s * DEG_SLAB, DEG_SLAB)],
                    degp_hbm.at[pl.ds(c * NPAD + s * DEG_SLAB, DEG_SLAB)])


WPW = NCHUNK // NW
PAIRS = WPW // 2
N_TAIL = N - (NS - 1) * ROW_SLAB


def _scatter_body(g_hbm, ei_hbm, zeros_hbm, accp_hbm,
                  sidx_flat, ebuf0, ebuf1, rows0, rows1,
                  acc, sem0, sem1, semd0, semd1):
    c = lax.axis_index("c")
    s = lax.axis_index("s")
    wid = s * NC + c
    base = wid * WPW * CH

    @pl.when(jnp.logical_and(c == 0, s < NS - 1))
    def _():
        pltpu.sync_copy(g_hbm.at[pl.ds(s * ROW_SLAB, ROW_SLAB)],
                        acc.at[pl.ds(s * ROW_SLAB, ROW_SLAB)])

    @pl.when(jnp.logical_and(c == 0, s == NS - 1))
    def _():
        pltpu.sync_copy(g_hbm.at[pl.ds((NS - 1) * ROW_SLAB, N_TAIL)],
                        acc.at[pl.ds((NS - 1) * ROW_SLAB, N_TAIL)])

    @pl.when(c == 1)
    def _():
        pltpu.sync_copy(zeros_hbm, acc.at[pl.ds(s * ROW_SLAB, ROW_SLAB)])

    pltpu.sync_copy(ei_hbm.at[0, pl.ds(base, WPW * CH)], sidx_flat)
    plsc.subcore_barrier()

    pltpu.async_copy(ei_hbm.at[:, pl.ds(base, CH)], ebuf0, semd0)
    pltpu.async_copy(g_hbm.at[sidx_flat.at[pl.ds(0, CH)]], rows0, sem0)

    def body(k, carry):
        t1 = 2 * k + 1
        pltpu.async_copy(ei_hbm.at[:, pl.ds(base + t1 * CH, CH)],
                         ebuf1, semd1)
        pltpu.async_copy(g_hbm.at[sidx_flat.at[pl.ds(t1 * CH, CH)]],
                         rows1, sem1)
        pltpu.make_async_copy(g_hbm.at[pl.ds(0, CH)], rows0, sem0).wait()
        pltpu.make_async_copy(ei_hbm.at[:, pl.ds(0, CH)], ebuf0,
                              semd0).wait()
        pltpu.sync_copy(rows0, acc.at[ebuf0.at[1]], add=True)

        @pl.when(2 * k + 2 < WPW)
        def _():
            pltpu.async_copy(ei_hbm.at[:, pl.ds(base + (2 * k + 2) * CH, CH)],
                             ebuf0, semd0)
            pltpu.async_copy(
                g_hbm.at[sidx_flat.at[pl.ds((2 * k + 2) * CH, CH)]],
                rows0, sem0)
        pltpu.make_async_copy(g_hbm.at[pl.ds(0, CH)], rows1, sem1).wait()
        pltpu.make_async_copy(ei_hbm.at[:, pl.ds(0, CH)], ebuf1,
                              semd1).wait()
        pltpu.sync_copy(rows1, acc.at[ebuf1.at[1]], add=True)
        return carry

    lax.fori_loop(0, PAIRS, body, 0)

    @pl.when(wid < NCHUNK - WPW * NW)
    def _():
        ex = WPW * NW + wid
        pltpu.sync_copy(ei_hbm.at[:, pl.ds(ex * CH, CH)], ebuf0)
        pltpu.sync_copy(ei_hbm.at[0, pl.ds(ex * CH, CH)],
                        sidx_flat.at[pl.ds(0, CH)])
        pltpu.async_copy(g_hbm.at[sidx_flat.at[pl.ds(0, CH)]],
                         rows0, sem0).wait()
        pltpu.sync_copy(rows0, acc.at[ebuf0.at[1]], add=True)

    plsc.subcore_barrier()
    pltpu.sync_copy(acc.at[pl.ds(s * ROW_SLAB, ROW_SLAB)],
                    accp_hbm.at[c, pl.ds(s * ROW_SLAB, ROW_SLAB)])


@functools.cache
def _sc_kernels():
    deg_kernel = pl.kernel(
        _deg_body,
        out_type=jax.ShapeDtypeStruct((NC * NPAD,), jnp.float32),
        mesh=_sc_mesh(),
        scratch_types=[
            pltpu.VMEM((2, CH), jnp.int32),
            pltpu.VMEM((2, CH), jnp.int32),
            pltpu.VMEM((2, CH), jnp.int32),
            pltpu.VMEM((2, CH), jnp.int32),
            pltpu.VMEM((CH,), jnp.float32),
            pltpu.VMEM_SHARED((NPAD,), jnp.float32),
            pltpu.SemaphoreType.DMA,
            pltpu.SemaphoreType.DMA,
            pltpu.SemaphoreType.DMA,
            pltpu.SemaphoreType.DMA,
        ],
    )
    scatter_kernel = pl.kernel(
        _scatter_body,
        out_type=jax.ShapeDtypeStruct((NC, NPAD, H), jnp.float32),
        mesh=_sc_mesh(),
        scratch_types=[
            pltpu.VMEM((WPW * CH,), jnp.int32),
            pltpu.VMEM((2, CH), jnp.int32),
            pltpu.VMEM((2, CH), jnp.int32),
            pltpu.VMEM((CH, H), jnp.float32),
            pltpu.VMEM((CH, H), jnp.float32),
            pltpu.VMEM_SHARED((NPAD, H), jnp.float32),
            pltpu.SemaphoreType.DMA,
            pltpu.SemaphoreType.DMA,
            pltpu.SemaphoreType.DMA,
            pltpu.SemaphoreType.DMA,
        ],
    )
    return deg_kernel, scatter_kernel


R = 2000
GRID = N // R


def _dinv_rows(deg_ref):
    deg = deg_ref[:, 0] + deg_ref[:, 1] + 1.0
    return lax.rsqrt(deg)[:, None]


def _tc_h0_body(x_ref, w0_ref, b0_ref, h_ref):
    h = jnp.dot(x_ref[...], w0_ref[...], preferred_element_type=jnp.float32)
    h_ref[...] = jnp.maximum(h + b0_ref[...], 0.0)


def _tc_g0_body(h_ref, deg_ref, c0_ref, g_ref):
    g_ref[...] = jnp.dot(_dinv_rows(deg_ref) * h_ref[...], c0_ref[...],
                         preferred_element_type=jnp.float32)


def _tc_post_body(acca_ref, accb_ref, deg_ref, b_ref,
                  z_ref, s1_ref, s2_ref):
    i = pl.program_id(0)
    z = _dinv_rows(deg_ref) * (acca_ref[0] + accb_ref[0])
    z = z + b_ref[...]
    z_ref[...] = z

    @pl.when(i == 0)
    def _():
        s1_ref[...] = jnp.zeros_like(s1_ref)
        s2_ref[...] = jnp.zeros_like(s2_ref)
    s1_ref[...] += jnp.sum(z, axis=0, keepdims=True)
    s2_ref[...] += jnp.sum(z * z, axis=0, keepdims=True)


def _bn_relu(z_ref, s1_ref, s2_ref, bg_ref, bb_ref):
    mean = s1_ref[...] * (1.0 / N)
    var = s2_ref[...] * (1.0 / N) - mean * mean
    zn = (z_ref[...] - mean) * lax.rsqrt(var + EPS) * bg_ref[...] + bb_ref[...]
    return jnp.maximum(zn, 0.0)


def _tc_mid_body(z_ref, s1_ref, s2_ref, bg_ref, bb_ref, deg_ref, c1_ref,
                 g_ref):
    h = _bn_relu(z_ref, s1_ref, s2_ref, bg_ref, bb_ref)
    g_ref[...] = jnp.dot(_dinv_rows(deg_ref) * h, c1_ref[...],
                         preferred_element_type=jnp.float32)


def _tc_out_body(z_ref, s1_ref, s2_ref, bg_ref, bb_ref, w_ref, b_ref, o_ref):
    h = _bn_relu(z_ref, s1_ref, s2_ref, bg_ref, bb_ref)
    o_ref[...] = jnp.dot(h, w_ref[...],
                         preferred_element_type=jnp.float32) + b_ref[...]


_row_spec = pl.BlockSpec((R, H), lambda i: (i, 0))
_w_spec = pl.BlockSpec((H, H), lambda i: (0, 0))
_v_spec = pl.BlockSpec((1, H), lambda i: (0, 0))
_deg_spec = pl.BlockSpec((R, 2), lambda i: (i, 0))

_f32 = jnp.float32

_tc_h0 = pl.pallas_call(
    _tc_h0_body,
    grid=(GRID,),
    in_specs=[_row_spec, _w_spec, _v_spec],
    out_specs=_row_spec,
    out_shape=jax.ShapeDtypeStruct((N, H), _f32),
)

_tc_g0 = pl.pallas_call(
    _tc_g0_body,
    grid=(GRID,),
    in_specs=[_row_spec, _deg_spec, _w_spec],
    out_specs=_row_spec,
    out_shape=jax.ShapeDtypeStruct((N, H), _f32),
)

_acc0_spec = pl.BlockSpec((1, R, H), lambda i: (0, i, 0))
_acc1_spec = pl.BlockSpec((1, R, H), lambda i: (1, i, 0))

_tc_post = pl.pallas_call(
    _tc_post_body,
    grid=(GRID,),
    in_specs=[_acc0_spec, _acc1_spec, _deg_spec, _v_spec],
    out_specs=[_row_spec, _v_spec, _v_spec],
    out_shape=[jax.ShapeDtypeStruct((N, H), _f32),
               jax.ShapeDtypeStruct((1, H), _f32),
               jax.ShapeDtypeStruct((1, H), _f32)],
)

_tc_mid = pl.pallas_call(
    _tc_mid_body,
    grid=(GRID,),
    in_specs=[_row_spec, _v_spec, _v_spec, _v_spec, _v_spec, _deg_spec,
              _w_spec],
    out_specs=_row_spec,
    out_shape=jax.ShapeDtypeStruct((N, H), _f32),
)

_tc_out = pl.pallas_call(
    _tc_out_body,
    grid=(GRID,),
    in_specs=[_row_spec, _v_spec, _v_spec, _v_spec, _v_spec, _w_spec,
              _v_spec],
    out_specs=_row_spec,
    out_shape=jax.ShapeDtypeStruct((N, H), _f32),
)


def kernel(x, edge_index, edge_attr, lin0_w, lin0_b, conv0_w, conv0_b,
           bn0_g, bn0_b, conv1_w, conv1_b, bn1_g, bn1_b, lin1_w, lin1_b):
    ei = edge_index.astype(jnp.int32)
    zeros_deg = jnp.zeros((DEG_SLAB,), _f32)
    zeros_acc = jnp.zeros((ROW_SLAB, H), _f32)

    _deg_kernel, _scatter_kernel = _sc_kernels()
    degp = _deg_kernel(ei, zeros_deg).reshape(NC, NPAD)
    deg2 = degp[:, :N].T

    b0 = lin0_b.reshape(1, H)
    cb0 = conv0_b.reshape(1, H)
    cb1 = conv1_b.reshape(1, H)
    h0 = _tc_h0(x, lin0_w, b0)
    g0 = _tc_g0(h0, deg2, conv0_w)

    accp0 = _scatter_kernel(g0, ei, zeros_acc)
    z0, s01, s02 = _tc_post(accp0, accp0, deg2, cb0)

    g1 = _tc_mid(z0, s01, s02, bn0_g.reshape(1, H), bn0_b.reshape(1, H),
                 deg2, conv1_w)

    accp1 = _scatter_kernel(g1, ei, zeros_acc)
    z1, s11, s12 = _tc_post(accp1, accp1, deg2, cb1)

    out = _tc_out(z1, s11, s12, bn1_g.reshape(1, H), bn1_b.reshape(1, H),
                  lin1_w, lin1_b.reshape(1, H))
    return out
